# Initial kernel scaffold; baseline (speedup 1.0000x reference)
import jax
import jax.numpy as jnp
from jax.experimental import pallas as pl


def kernel(x, gt):
    raise NotImplementedError("write your pallas kernel here")


# speedup vs baseline: 5.1031x; 5.1031x over previous
import functools

import jax
import jax.numpy as jnp
import numpy as np
from jax.experimental import pallas as pl
from jax.experimental.pallas import tpu as pltpu

_SIZE = 32000
_PADDING_IDX = 0
_SMOOTHING = 0.1
_CONFIDENCE = 1.0 - _SMOOTHING
_EPS = np.float32(_SMOOTHING / (_SIZE - 2))
_C1 = np.float32((_SIZE - 2) * float(_EPS) * np.log(float(_EPS)))
_CLOGC = np.float32(_CONFIDENCE * np.log(_CONFIDENCE))
_CONF_F = np.float32(_CONFIDENCE)


def _loss_body(x_ref, gt_ref, out_ref, m_ref, idx_ref, s_ref, xat_ref, x0_ref,
               *, n_col_blocks, n_row_blocks, inv_count):
    j = pl.program_id(1)
    i = pl.program_id(0)

    x_tile = x_ref[...]
    gt_tile = gt_ref[...]
    cols = x_tile.shape[1]
    col_base = j * cols

    tm = jnp.max(gt_tile, axis=1, keepdims=True)
    ta = jnp.argmax(gt_tile, axis=1)[:, None] + col_base
    onehot = jax.lax.broadcasted_iota(jnp.int32, gt_tile.shape, 1) == (ta - col_base)
    txat = jnp.sum(jnp.where(onehot, x_tile, 0.0), axis=1, keepdims=True)
    ts = jnp.sum(x_tile, axis=1, keepdims=True)

    @pl.when(j == 0)
    def _init():
        m_ref[...] = tm
        idx_ref[...] = ta
        s_ref[...] = ts
        xat_ref[...] = txat
        x0_ref[...] = x_tile[:, 0:1]

    @pl.when(j != 0)
    def _update():
        better = tm > m_ref[...]
        m_ref[...] = jnp.where(better, tm, m_ref[...])
        idx_ref[...] = jnp.where(better, ta, idx_ref[...])
        xat_ref[...] = jnp.where(better, txat, xat_ref[...])
        s_ref[...] = s_ref[...] + ts

    @pl.when(j == n_col_blocks - 1)
    def _finish():
        idx = idx_ref[...]
        s = s_ref[...]
        xat = xat_ref[...]
        x0 = x0_ref[...]
        contrib = _C1 - _EPS * (s - x0 - xat) + (_CLOGC - _CONF_F * xat)
        contrib = jnp.where(idx == _PADDING_IDX, 0.0, contrib)
        total = jnp.reshape(jnp.sum(contrib) * inv_count, (1, 1))

        @pl.when(i == 0)
        def _():
            out_ref[...] = total

        @pl.when(i != 0)
        def _():
            out_ref[...] = out_ref[...] + total


@jax.jit
def kernel(x, gt):
    n, size = x.shape
    row_block = 512
    col_block = 3200
    n_row_blocks = n // row_block
    n_col_blocks = size // col_block
    inv_count = np.float32(1.0 / (n * size))

    body = functools.partial(
        _loss_body,
        n_col_blocks=n_col_blocks,
        n_row_blocks=n_row_blocks,
        inv_count=inv_count,
    )

    out = pl.pallas_call(
        body,
        grid=(n_row_blocks, n_col_blocks),
        in_specs=[
            pl.BlockSpec((row_block, col_block), lambda i, j: (i, j)),
            pl.BlockSpec((row_block, col_block), lambda i, j: (i, j)),
        ],
        out_specs=pl.BlockSpec((1, 1), lambda i, j: (0, 0)),
        out_shape=jax.ShapeDtypeStruct((1, 1), jnp.float32),
        scratch_shapes=[
            pltpu.VMEM((row_block, 1), jnp.float32),
            pltpu.VMEM((row_block, 1), jnp.int32),
            pltpu.VMEM((row_block, 1), jnp.float32),
            pltpu.VMEM((row_block, 1), jnp.float32),
            pltpu.VMEM((row_block, 1), jnp.float32),
        ],
    )(x, gt)
    return out[0, 0]
